# baseline (speedup 1.0000x reference)
import dataclasses

import jax
import jax.numpy as jnp
from jax import lax
from jax.experimental import pallas as pl
from jax.experimental.pallas import tpu as pltpu
from jax.experimental.pallas import tpu_sc as plsc

EMB_DIM = 128
BATCH = 16384

NC, NS, L = 2, 16, 16
NW = NC * NS
B_PER_W = BATCH // NW
CHUNK = 128
N_CHUNKS = B_PER_W // CHUNK


def _mfcf_sc(u, i, U, I):
    mesh = plsc.VectorSubcoreMesh(core_axis_name="c", subcore_axis_name="s")
    cp = pltpu.CompilerParams()
    if "needs_layout_passes" in pltpu.CompilerParams.__dataclass_fields__:
        cp = dataclasses.replace(cp, needs_layout_passes=False)

    @pl.kernel(
        compiler_params=cp,
        out_type=jax.ShapeDtypeStruct((BATCH,), jnp.float32),
        mesh=mesh,
        scratch_types=[
            pltpu.VMEM((B_PER_W,), jnp.int32),
            pltpu.VMEM((B_PER_W,), jnp.int32),
            pltpu.VMEM((2 * CHUNK, EMB_DIM), jnp.float32),
            pltpu.VMEM((2 * CHUNK, EMB_DIM), jnp.float32),
            pltpu.VMEM((B_PER_W,), jnp.float32),
            pltpu.VMEM((L * (L + 1),), jnp.float32),
            pltpu.SemaphoreType.DMA,
            pltpu.SemaphoreType.DMA,
            pltpu.SemaphoreType.DMA,
            pltpu.SemaphoreType.DMA,
            pltpu.SemaphoreType.DMA,
        ],
    )
    def kern(u_hbm, i_hbm, U_hbm, I_hbm, o_hbm,
             uidx, iidx, uebuf, iebuf, outv, tbuf,
             semu0, semu1, semi0, semi1, semo):
        wid = lax.axis_index("s") * NC + lax.axis_index("c")
        base = wid * B_PER_W

        def start(c, roff):
            pltpu.async_copy(
                U_hbm.at[uidx.at[pl.ds(c * CHUNK, CHUNK)]],
                uebuf.at[pl.ds(roff, CHUNK)], semu0)
            pltpu.async_copy(
                I_hbm.at[iidx.at[pl.ds(c * CHUNK, CHUNK)]],
                iebuf.at[pl.ds(roff, CHUNK)], semi0)

        tbase = lax.broadcasted_iota(jnp.int32, (L,), 0) * (L + 1)

        cu0 = pltpu.async_copy(
            u_hbm.at[pl.ds(base, CHUNK)], uidx.at[pl.ds(0, CHUNK)], semu1)
        ci0 = pltpu.async_copy(
            i_hbm.at[pl.ds(base, CHUNK)], iidx.at[pl.ds(0, CHUNK)], semi1)
        cu0.wait()
        ci0.wait()
        start(0, 0)
        rest = B_PER_W - CHUNK
        pltpu.sync_copy(u_hbm.at[pl.ds(base + CHUNK, rest)],
                        uidx.at[pl.ds(CHUNK, rest)])
        pltpu.sync_copy(i_hbm.at[pl.ds(base + CHUNK, rest)],
                        iidx.at[pl.ds(CHUNK, rest)])

        @pl.loop(0, N_CHUNKS)
        def _(c):
            roff = (c & 1) * CHUNK
            pltpu.make_async_copy(
                U_hbm.at[pl.ds(0, CHUNK)],
                uebuf.at[pl.ds(0, CHUNK)], semu0).wait()
            pltpu.make_async_copy(
                I_hbm.at[pl.ds(0, CHUNK)],
                iebuf.at[pl.ds(0, CHUNK)], semi0).wait()

            @pl.when(c + 1 < N_CHUNKS)
            def _():
                start(c + 1, CHUNK - roff)

            def load_row(r):
                return [(uebuf[r, pl.ds(cc * L, L)],
                         iebuf[r, pl.ds(cc * L, L)])
                        for cc in range(EMB_DIM // L)]

            @pl.loop(0, CHUNK // L)
            def _(g):
                rb = roff + g * L
                lds = load_row(rb)
                for j in range(L):
                    nxt = load_row(rb + j + 1) if j + 1 < L else None
                    p = [a * b for a, b in lds]
                    while len(p) > 1:
                        p = [a + b for a, b in zip(p[::2], p[1::2])]
                    tbuf[pl.ds(j * (L + 1), L)] = p[0]
                    lds = nxt
                cols = [plsc.load_gather(tbuf, [tbase + l]) for l in range(L)]
                while len(cols) > 1:
                    cols = [a + b for a, b in zip(cols[::2], cols[1::2])]
                sig = 1.0 / (1.0 + jnp.exp(-cols[0]))
                outv[pl.ds(c * CHUNK + g * L, L)] = sig

            pltpu.async_copy(
                outv.at[pl.ds(c * CHUNK, CHUNK)],
                o_hbm.at[pl.ds(base + c * CHUNK, CHUNK)], semo)

        for _c in range(N_CHUNKS):
            pltpu.make_async_copy(
                outv.at[pl.ds(0, CHUNK)],
                o_hbm.at[pl.ds(base, CHUNK)], semo).wait()

    return kern(u, i, U, I)


def kernel(u, i, U, I):
    u = u.astype(jnp.int32)
    i = i.astype(jnp.int32)
    U = U.astype(jnp.float32)
    I = I.astype(jnp.float32)
    return _mfcf_sc(u, i, U, I)


# speedup vs baseline: 1.0136x; 1.0136x over previous
import dataclasses

import jax
import jax.numpy as jnp
from jax import lax
from jax.experimental import pallas as pl
from jax.experimental.pallas import tpu as pltpu
from jax.experimental.pallas import tpu_sc as plsc

EMB_DIM = 128
BATCH = 16384

NC, NS, L = 2, 16, 16
NW = NC * NS
B_PER_W = BATCH // NW
CHUNK = 128
N_CHUNKS = B_PER_W // CHUNK


def _mfcf_sc(u, i, U, I):
    mesh = plsc.VectorSubcoreMesh(core_axis_name="c", subcore_axis_name="s")
    cp = pltpu.CompilerParams()
    if "needs_layout_passes" in pltpu.CompilerParams.__dataclass_fields__:
        cp = dataclasses.replace(cp, needs_layout_passes=False)

    @pl.kernel(
        compiler_params=cp,
        out_type=jax.ShapeDtypeStruct((BATCH,), jnp.float32),
        mesh=mesh,
        scratch_types=[
            pltpu.VMEM((B_PER_W,), jnp.int32),
            pltpu.VMEM((B_PER_W,), jnp.int32),
            pltpu.VMEM((CHUNK, EMB_DIM), jnp.float32),
            pltpu.VMEM((CHUNK, EMB_DIM), jnp.float32),
            pltpu.VMEM((CHUNK, EMB_DIM), jnp.float32),
            pltpu.VMEM((CHUNK, EMB_DIM), jnp.float32),
            pltpu.VMEM((B_PER_W,), jnp.float32),
            pltpu.VMEM((L * (L + 1),), jnp.float32),
            pltpu.SemaphoreType.DMA,
            pltpu.SemaphoreType.DMA,
            pltpu.SemaphoreType.DMA,
            pltpu.SemaphoreType.DMA,
            pltpu.SemaphoreType.DMA,
        ],
    )
    def kern(u_hbm, i_hbm, U_hbm, I_hbm, o_hbm,
             uidx, iidx, ue0, ue1, ie0, ie1, outv, tbuf,
             semu0, semu1, semi0, semi1, semo):
        wid = lax.axis_index("s") * NC + lax.axis_index("c")
        base = wid * B_PER_W

        ue_bufs = (ue0, ue1)
        ie_bufs = (ie0, ie1)
        semu = (semu0, semu1)
        semi = (semi0, semi1)

        def start(c):
            b = c % 2
            cu = pltpu.async_copy(
                U_hbm.at[uidx.at[pl.ds(c * CHUNK, CHUNK)]], ue_bufs[b], semu[b])
            ci = pltpu.async_copy(
                I_hbm.at[iidx.at[pl.ds(c * CHUNK, CHUNK)]], ie_bufs[b], semi[b])
            return cu, ci

        tbase = lax.broadcasted_iota(jnp.int32, (L,), 0) * (L + 1)

        cu0 = pltpu.async_copy(
            u_hbm.at[pl.ds(base, CHUNK)], uidx.at[pl.ds(0, CHUNK)], semu0)
        ci0 = pltpu.async_copy(
            i_hbm.at[pl.ds(base, CHUNK)], iidx.at[pl.ds(0, CHUNK)], semi0)
        cu0.wait()
        gu0 = pltpu.async_copy(
            U_hbm.at[uidx.at[pl.ds(0, CHUNK)]], ue_bufs[0], semu[0])
        ci0.wait()
        gi0 = pltpu.async_copy(
            I_hbm.at[iidx.at[pl.ds(0, CHUNK)]], ie_bufs[0], semi[0])
        inflight = (gu0, gi0)
        rest = B_PER_W - CHUNK
        pltpu.sync_copy(u_hbm.at[pl.ds(base + CHUNK, rest)],
                        uidx.at[pl.ds(CHUNK, rest)])
        pltpu.sync_copy(i_hbm.at[pl.ds(base + CHUNK, rest)],
                        iidx.at[pl.ds(CHUNK, rest)])

        start(1)

        def compute_chunk(ue, ie, c):
            def load_row(r):
                return [(ue[r, pl.ds(cc * L, L)], ie[r, pl.ds(cc * L, L)])
                        for cc in range(EMB_DIM // L)]

            @pl.loop(0, CHUNK // L)
            def _(g):
                lds = load_row(g * L)
                for j in range(L):
                    nxt = load_row(g * L + j + 1) if j + 1 < L else None
                    p = [a * b for a, b in lds]
                    while len(p) > 1:
                        p = [a + b for a, b in zip(p[::2], p[1::2])]
                    tbuf[pl.ds(j * (L + 1), L)] = p[0]
                    lds = nxt
                cols = [plsc.load_gather(tbuf, [tbase + l]) for l in range(L)]
                while len(cols) > 1:
                    cols = [a + b for a, b in zip(cols[::2], cols[1::2])]
                sig = 1.0 / (1.0 + jnp.exp(-cols[0]))
                outv[pl.ds(c * CHUNK + g * L, L)] = sig

        @pl.loop(0, N_CHUNKS // 2)
        def _(o):
            for par in range(2):
                c = 2 * o + par
                pltpu.make_async_copy(
                    U_hbm.at[pl.ds(0, CHUNK)], ue_bufs[par], semu[par]).wait()
                pltpu.make_async_copy(
                    I_hbm.at[pl.ds(0, CHUNK)], ie_bufs[par], semi[par]).wait()
                compute_chunk(ue_bufs[par], ie_bufs[par], c)
                pltpu.async_copy(
                    outv.at[pl.ds(c * CHUNK, CHUNK)],
                    o_hbm.at[pl.ds(base + c * CHUNK, CHUNK)], semo)

                @pl.when(o + 1 < N_CHUNKS // 2)
                def _():
                    start(2 + par)

        for _c in range(N_CHUNKS):
            pltpu.make_async_copy(
                outv.at[pl.ds(0, CHUNK)],
                o_hbm.at[pl.ds(base, CHUNK)], semo).wait()

    return kern(u, i, U, I)


def kernel(u, i, U, I):
    u = u.astype(jnp.int32)
    i = i.astype(jnp.int32)
    U = U.astype(jnp.float32)
    I = I.astype(jnp.float32)
    return _mfcf_sc(u, i, U, I)


# speedup vs baseline: 1.0140x; 1.0004x over previous
import dataclasses

import jax
import jax.numpy as jnp
from jax import lax
from jax.experimental import pallas as pl
from jax.experimental.pallas import tpu as pltpu
from jax.experimental.pallas import tpu_sc as plsc

EMB_DIM = 128
BATCH = 16384

NC, NS, L = 2, 16, 16
NW = NC * NS
B_PER_W = BATCH // NW
CHUNK = 128
N_CHUNKS = B_PER_W // CHUNK


def _mfcf_sc(u, i, U, I):
    mesh = plsc.VectorSubcoreMesh(core_axis_name="c", subcore_axis_name="s")
    cp = pltpu.CompilerParams()
    if "needs_layout_passes" in pltpu.CompilerParams.__dataclass_fields__:
        cp = dataclasses.replace(cp, needs_layout_passes=False)

    @pl.kernel(
        compiler_params=cp,
        out_type=jax.ShapeDtypeStruct((BATCH,), jnp.float32),
        mesh=mesh,
        scratch_types=[
            pltpu.VMEM((B_PER_W,), jnp.int32),
            pltpu.VMEM((B_PER_W,), jnp.int32),
            pltpu.VMEM((CHUNK, EMB_DIM), jnp.float32),
            pltpu.VMEM((CHUNK, EMB_DIM), jnp.float32),
            pltpu.VMEM((CHUNK, EMB_DIM), jnp.float32),
            pltpu.VMEM((CHUNK, EMB_DIM), jnp.float32),
            pltpu.VMEM((B_PER_W,), jnp.float32),
            pltpu.VMEM((L * (L + 1),), jnp.float32),
            pltpu.SemaphoreType.DMA,
            pltpu.SemaphoreType.DMA,
            pltpu.SemaphoreType.DMA,
            pltpu.SemaphoreType.DMA,
            pltpu.SemaphoreType.DMA,
        ],
    )
    def kern(u_hbm, i_hbm, U_hbm, I_hbm, o_hbm,
             uidx, iidx, ue0, ue1, ie0, ie1, outv, tbuf,
             semu0, semu1, semi0, semi1, semo):
        wid = lax.axis_index("s") * NC + lax.axis_index("c")
        base = wid * B_PER_W

        ue_bufs = (ue0, ue1)
        ie_bufs = (ie0, ie1)
        semu = (semu0, semu1)
        semi = (semi0, semi1)

        def start(c):
            b = c % 2
            cu = pltpu.async_copy(
                U_hbm.at[uidx.at[pl.ds(c * CHUNK, CHUNK)]], ue_bufs[b], semu[b])
            ci = pltpu.async_copy(
                I_hbm.at[iidx.at[pl.ds(c * CHUNK, CHUNK)]], ie_bufs[b], semi[b])
            return cu, ci

        tbase = lax.broadcasted_iota(jnp.int32, (L,), 0) * (L + 1)

        cu0 = pltpu.async_copy(
            u_hbm.at[pl.ds(base, CHUNK)], uidx.at[pl.ds(0, CHUNK)], semu0)
        ci0 = pltpu.async_copy(
            i_hbm.at[pl.ds(base, CHUNK)], iidx.at[pl.ds(0, CHUNK)], semi0)
        cu0.wait()
        gu0 = pltpu.async_copy(
            U_hbm.at[uidx.at[pl.ds(0, CHUNK)]], ue_bufs[0], semu[0])
        ci0.wait()
        gi0 = pltpu.async_copy(
            I_hbm.at[iidx.at[pl.ds(0, CHUNK)]], ie_bufs[0], semi[0])
        del gu0, gi0
        rest = B_PER_W - CHUNK
        pltpu.sync_copy(u_hbm.at[pl.ds(base + CHUNK, rest)],
                        uidx.at[pl.ds(CHUNK, rest)])
        pltpu.sync_copy(i_hbm.at[pl.ds(base + CHUNK, rest)],
                        iidx.at[pl.ds(CHUNK, rest)])

        start(1)

        def compute_chunk(ue, ie, c):
            def load_row(r):
                return [(ue[r, pl.ds(cc * L, L)], ie[r, pl.ds(cc * L, L)])
                        for cc in range(EMB_DIM // L)]

            @pl.loop(0, CHUNK // L)
            def _(g):
                lds = load_row(g * L)
                for j in range(L):
                    nxt = load_row(g * L + j + 1) if j + 1 < L else None
                    p = [a * b for a, b in lds]
                    while len(p) > 1:
                        p = [a + b for a, b in zip(p[::2], p[1::2])]
                    tbuf[pl.ds(j * (L + 1), L)] = p[0]
                    lds = nxt
                cols = [plsc.load_gather(tbuf, [tbase + l]) for l in range(L)]
                while len(cols) > 1:
                    cols = [a + b for a, b in zip(cols[::2], cols[1::2])]
                sig = 1.0 / (1.0 + jnp.exp(-cols[0]))
                outv[pl.ds(c * CHUNK + g * L, L)] = sig

        @pl.loop(0, N_CHUNKS // 2)
        def _(o):
            for par in range(2):
                c = 2 * o + par
                pltpu.make_async_copy(
                    U_hbm.at[pl.ds(0, CHUNK)], ue_bufs[par], semu[par]).wait()
                pltpu.make_async_copy(
                    I_hbm.at[pl.ds(0, CHUNK)], ie_bufs[par], semi[par]).wait()
                compute_chunk(ue_bufs[par], ie_bufs[par], c)
                pltpu.async_copy(
                    outv.at[pl.ds(c * CHUNK, CHUNK)],
                    o_hbm.at[pl.ds(base + c * CHUNK, CHUNK)], semo)

                @pl.when(o + 1 < N_CHUNKS // 2)
                def _():
                    start(2 + par)

        for _c in range(N_CHUNKS):
            pltpu.make_async_copy(
                outv.at[pl.ds(0, CHUNK)],
                o_hbm.at[pl.ds(base, CHUNK)], semo).wait()

    return kern(u, i, U, I)


def kernel(u, i, U, I):
    u = u.astype(jnp.int32)
    i = i.astype(jnp.int32)
    U = U.astype(jnp.float32)
    I = I.astype(jnp.float32)
    return _mfcf_sc(u, i, U, I)
